# Initial kernel scaffold; baseline (speedup 1.0000x reference)
import jax
import jax.numpy as jnp
from jax.experimental import pallas as pl


def kernel(steps, embedding):
    raise NotImplementedError("write your pallas kernel here")


# speedup vs baseline: 2.0163x; 2.0163x over previous
import functools

import jax
import jax.numpy as jnp
from jax import lax
from jax.experimental import pallas as pl
from jax.experimental.pallas import tpu as pltpu
from jax.experimental.pallas import tpu_sc as plsc

_D = 110
_DP = 112
_CHUNK = 128


@functools.lru_cache(maxsize=None)
def _build_gather(B: int, V: int, D: int):
    info = plsc.get_sparse_core_info()
    NC, NS = info.num_cores, info.num_subcores
    NW = NC * NS
    assert B % (NW * _CHUNK) == 0
    b_per_w = B // NW
    n_chunks = b_per_w // _CHUNK
    mesh = plsc.VectorSubcoreMesh(core_axis_name="c", subcore_axis_name="s")

    @functools.partial(
        pl.kernel,
        out_type=jax.ShapeDtypeStruct((B, _DP), jnp.float32),
        mesh=mesh,
        scratch_types=[
            pltpu.VMEM((n_chunks, _CHUNK), jnp.int32),
            pltpu.VMEM((_CHUNK, _DP), jnp.float32),
            pltpu.SemaphoreType.DMA,
        ],
        compiler_params=pltpu.CompilerParams(use_tc_tiling_on_sc=False),
    )
    def gather(steps_hbm, table_hbm, out_hbm, idx_v, rows_v, sem):
        wid = lax.axis_index("s") * NC + lax.axis_index("c")
        base = wid * b_per_w
        pltpu.sync_copy(steps_hbm.at[pl.ds(wid * n_chunks, n_chunks)], idx_v)

        def chunk_body(c, carry):
            off = pl.multiple_of(c * _CHUNK, _CHUNK)
            pltpu.async_copy(table_hbm.at[idx_v.at[c]], rows_v, sem).wait()
            pltpu.sync_copy(rows_v, out_hbm.at[pl.ds(base + off, _CHUNK)])
            return carry

        lax.fori_loop(0, n_chunks, chunk_body, 0)

    return gather


def kernel(steps, embedding):
    B = steps.shape[0] * steps.shape[1]
    V, D = embedding.shape
    flat = steps.reshape(B // _CHUNK, _CHUNK)
    emb_p = jnp.pad(embedding, ((0, 0), (0, _DP - D)))
    out = _build_gather(B, V, D)(flat, emb_p)
    return out[:, :D].reshape(steps.shape[0], steps.shape[1], D)
